# baseline (speedup 1.0000x reference)
import functools

import numpy as np
import jax
import jax.numpy as jnp
from jax import lax
from jax.experimental import pallas as pl
from jax.experimental.pallas import tpu as pltpu
from jax.experimental.pallas import tpu_sc as plsc

_VOCAB = 1000000
_D = 64
_B = 32
_S = 2048

_NC = 2
_NS = 16
_NW = _NC * _NS
_CHUNK = _S // _NW
_LANES = 16


def _positional_encoding(seq_len, d_model):
    pos = np.arange(seq_len, dtype=np.float64)[:, None]
    i = np.arange(0, d_model, 2, dtype=np.float64)
    angle = pos / (10000.0 ** (2.0 * i / d_model))
    pe = np.zeros((seq_len, d_model), dtype=np.float64)
    pe[:, 0::2] = np.sin(angle)
    pe[:, 1::2] = np.cos(angle)
    return jnp.asarray(pe, dtype=jnp.float32)


def _build_kernel():
    mesh = plsc.VectorSubcoreMesh(
        core_axis_name="c", subcore_axis_name="s",
        num_cores=_NC, num_subcores=_NS,
    )

    @functools.partial(
        pl.kernel,
        mesh=mesh,
        compiler_params=pltpu.CompilerParams(use_tc_tiling_on_sc=False),
        out_type=jax.ShapeDtypeStruct((_B, _S, _D), jnp.float32),
        scratch_types=[
            pltpu.VMEM((_B, _CHUNK), jnp.int32),
            pltpu.VMEM((_CHUNK, _D), jnp.float32),
            pltpu.VMEM((2, _CHUNK, _D), jnp.float32),
            pltpu.SemaphoreType.DMA((2,)),
            pltpu.SemaphoreType.DMA,
        ],
    )
    def emb_kernel(x_h, table_h, pe_h, out_h, idx_v, pe_v, buf, sems, isem):
        wid = lax.axis_index("s") * _NC + lax.axis_index("c")
        base = wid * _CHUNK

        pltpu.sync_copy(pe_h.at[pl.ds(base, _CHUNK), :], pe_v)
        idx_copies = [
            pltpu.async_copy(
                x_h.at[pl.ds(b * _S + base, _CHUNK)], idx_v.at[b], isem)
            for b in range(_B)
        ]
        for cp in idx_copies:
            cp.wait()

        copies = [None, None]
        copies[0] = pltpu.async_copy(
            table_h.at[idx_v.at[0]], buf.at[0], sems.at[0])

        for b in range(_B):
            slot = b % 2
            copies[slot].wait()
            if b + 1 < _B:
                nxt = (b + 1) % 2
                copies[nxt] = pltpu.async_copy(
                    table_h.at[idx_v.at[b + 1]], buf.at[nxt], sems.at[nxt])

            def add_row(r, _):
                for j in range(_D // _LANES):
                    sl = pl.ds(j * _LANES, _LANES)
                    buf[slot, r, sl] = buf[slot, r, sl] + pe_v[r, sl]
                return 0

            lax.fori_loop(0, _CHUNK, add_row, 0)
            pltpu.sync_copy(buf.at[slot], out_h.at[b, pl.ds(base, _CHUNK), :])

    return emb_kernel


_EMB_KERNEL = None


def kernel(x, table):
    global _EMB_KERNEL
    if _EMB_KERNEL is None:
        _EMB_KERNEL = _build_kernel()
    pe = _positional_encoding(_S, _D)
    return _EMB_KERNEL(jnp.reshape(x, (_B * _S,)), table, pe)
